# baseline (speedup 1.0000x reference)
import functools

import jax
import jax.numpy as jnp
from jax import lax
from jax.experimental import pallas as pl
from jax.experimental.pallas import tpu as pltpu
from jax.experimental.pallas import tpu_sc as plsc

BS = 8
P = 196
D = 384
T = 512
K = 20
NUM_CLASSES = 21
PS = 16
NPIX = PS * PS
NROWS = 14
G = 7

NPAIR = P * BS
NW = 32
PAIRS_PER_W = NPAIR // NW
L = 16


def _tc_body(tf_ref, trf_ref, lab_ref, labc_ref, wexp_ref):
    tf = tf_ref[...]
    trf = trf_ref[...]
    sim = lax.dot_general(
        tf, trf, (((2,), (1,)), ((0,), (0,))),
        preferred_element_type=jnp.float32,
    ).reshape(G * BS, T)

    cur = sim
    iota = lax.broadcasted_iota(jnp.int32, (G * BS, T), 1)
    masks = []
    ms = []
    for _ in range(K):
        m = jnp.max(cur, axis=1, keepdims=True)
        e = cur == m
        first = jnp.min(jnp.where(e, iota, T), axis=1, keepdims=True)
        e = iota == first
        masks.append(e.astype(jnp.float32))
        ms.append(m)
        cur = jnp.where(e, -jnp.inf, cur)
    S = jnp.stack(masks, axis=1)
    mk = jnp.concatenate(ms, axis=1)
    w = jax.nn.softmax(mk, axis=1)

    labf = lab_ref[...].astype(jnp.float32)
    labc = lax.dot_general(
        S.reshape(G, BS * K, T), labf, (((2,), (2,)), ((0,), (0,))),
        preferred_element_type=jnp.float32,
    )
    labc_ref[...] = labc.astype(jnp.int32)
    wexp_ref[...] = w.reshape(G, BS, K)


def _sc_vote(labc_hbm, wexp_hbm, out_hbm, lab_v, w_v, acc_v, pred_v):
    wid = lax.axis_index("s") * 2 + lax.axis_index("c")
    lane = lax.broadcasted_iota(jnp.int32, (L,), 0)

    def pair_body(i, carry):
        pair = wid * PAIRS_PER_W + i
        pltpu.sync_copy(labc_hbm.at[pair], lab_v)
        pltpu.sync_copy(wexp_hbm.at[pair], w_v)
        zero = jnp.zeros((L,), jnp.float32)
        for j in range(NUM_CLASSES * NPIX // L):
            acc_v[pl.ds(j * L, L)] = zero
        for k in range(K):
            wk = w_v[k]
            for c in range(NPIX // L):
                labk = lab_v[k, pl.ds(c * L, L)]
                idx = labk * NPIX + (c * L + lane)
                plsc.addupdate_scatter(acc_v, [idx], wk)
        for c in range(NPIX // L):
            best_v = jnp.full((L,), -1.0, jnp.float32)
            best_c = jnp.zeros((L,), jnp.int32)
            for cls in range(NUM_CLASSES):
                v = acc_v[pl.ds(cls * NPIX + c * L, L)]
                upd = v > best_v
                best_v = jnp.where(upd, v, best_v)
                best_c = jnp.where(upd, cls, best_c)
            pred_v[pl.ds(c * L, L)] = best_c
        pltpu.sync_copy(pred_v, out_hbm.at[pair])
        return carry

    lax.fori_loop(0, PAIRS_PER_W, pair_body, 0)


@functools.partial(
    pl.kernel,
    out_type=jax.ShapeDtypeStruct((NPAIR, NPIX), jnp.int32),
    mesh=plsc.VectorSubcoreMesh(core_axis_name="c", subcore_axis_name="s"),
    compiler_params=pltpu.CompilerParams(needs_layout_passes=False),
    scratch_types=[
        pltpu.VMEM((K, NPIX), jnp.int32),
        pltpu.VMEM((K, L), jnp.float32),
        pltpu.VMEM((NUM_CLASSES * NPIX,), jnp.float32),
        pltpu.VMEM((NPIX,), jnp.int32),
    ],
)
def _sc_vote_kernel(labc_hbm, wexp_hbm, out_hbm, lab_v, w_v, acc_v, pred_v):
    _sc_vote(labc_hbm, wexp_hbm, out_hbm, lab_v, w_v, acc_v, pred_v)


def kernel(test_feature, train_features, train_labels):
    tf_t = jnp.transpose(test_feature, (1, 0, 2))
    labc, wexp = pl.pallas_call(
        _tc_body,
        grid=(P // G,),
        in_specs=[
            pl.BlockSpec((G, BS, D), lambda p: (p, 0, 0)),
            pl.BlockSpec((G, D, T), lambda p: (p, 0, 0)),
            pl.BlockSpec((G, NPIX, T), lambda p: (p, 0, 0)),
        ],
        out_specs=[
            pl.BlockSpec((G, BS * K, NPIX), lambda p: (p, 0, 0)),
            pl.BlockSpec((G, BS, K), lambda p: (p, 0, 0)),
        ],
        out_shape=[
            jax.ShapeDtypeStruct((P, BS * K, NPIX), jnp.int32),
            jax.ShapeDtypeStruct((P, BS, K), jnp.float32),
        ],
    )(tf_t, train_features, train_labels)

    wexp = jnp.broadcast_to(
        wexp.reshape(NPAIR, K, 1), (NPAIR, K, L)).astype(jnp.float32)
    pred = _sc_vote_kernel(labc.reshape(NPAIR, K, NPIX), wexp)

    img = jnp.transpose(pred.reshape(P, BS, NPIX), (1, 0, 2))
    img = img.reshape(BS, NROWS, NROWS, PS, PS)
    img = jnp.transpose(img, (0, 1, 3, 2, 4)).reshape(BS, NROWS * PS, NROWS * PS)
    return img


# speedup vs baseline: 1.1261x; 1.1261x over previous
import functools

import jax
import jax.numpy as jnp
from jax import lax
from jax.experimental import pallas as pl
from jax.experimental.pallas import tpu as pltpu
from jax.experimental.pallas import tpu_sc as plsc

BS = 8
P = 196
D = 384
T = 512
K = 20
NUM_CLASSES = 21
PS = 16
NPIX = PS * PS
NROWS = 14
G = 7

NPAIR = P * BS
NW = 32
PAIRS_PER_W = NPAIR // NW
L = 16


def _tc_body(tf_ref, trf_ref, lab_ref, labc_ref, wexp_ref):
    tf = tf_ref[...]
    trf = trf_ref[...]
    sim = lax.dot_general(
        tf, trf, (((2,), (1,)), ((0,), (0,))),
        preferred_element_type=jnp.float32,
    ).reshape(G * BS, T)

    cur = sim
    iota = lax.broadcasted_iota(jnp.int32, (G * BS, T), 1)
    masks = []
    ms = []
    for _ in range(K):
        m = jnp.max(cur, axis=1, keepdims=True)
        e = cur == m
        first = jnp.min(jnp.where(e, iota, T), axis=1, keepdims=True)
        e = iota == first
        masks.append(e.astype(jnp.float32))
        ms.append(m)
        cur = jnp.where(e, -jnp.inf, cur)
    S = jnp.stack(masks, axis=1)
    mk = jnp.concatenate(ms, axis=1)
    w = jax.nn.softmax(mk, axis=1)

    labf = lab_ref[...].astype(jnp.float32)
    labc = lax.dot_general(
        S.reshape(G, BS * K, T), labf, (((2,), (2,)), ((0,), (0,))),
        preferred_element_type=jnp.float32,
    )
    labc_ref[...] = labc.astype(jnp.int32)
    wexp_ref[...] = w.reshape(G, BS, K)


def _sc_vote(labc_hbm, wflat_hbm, out_hbm, lab_v, w_v, acc_v, pred_v,
             in_sem, out_sem):
    wid = lax.axis_index("s") * 2 + lax.axis_index("c")
    lane = lax.broadcasted_iota(jnp.int32, (L,), 0)
    base = wid * PAIRS_PER_W

    def start_in(i, buf):
        pltpu.async_copy(labc_hbm.at[base + i], lab_v.at[buf], in_sem)
        pltpu.async_copy(wflat_hbm.at[base + i], w_v.at[buf], in_sem)

    def drain_in(buf):
        pltpu.make_async_copy(labc_hbm.at[base], lab_v.at[buf], in_sem).wait()
        pltpu.make_async_copy(wflat_hbm.at[base], w_v.at[buf], in_sem).wait()

    start_in(0, 0)

    def process(i, buf):
        drain_in(buf)
        start_in(jnp.minimum(i + 1, PAIRS_PER_W - 1), 1 - buf)

        zero = jnp.zeros((L,), jnp.float32)
        for j in range(NUM_CLASSES * NPIX // L):
            acc_v[pl.ds(j * L, L)] = zero
        for k in range(K):
            wk = w_v[buf, pl.ds(k * L, L)]
            for c in range(NPIX // L):
                labk = lab_v[buf, k, pl.ds(c * L, L)]
                idx = labk * NPIX + (c * L + lane)
                plsc.addupdate_scatter(acc_v, [idx], wk)
        for c in range(NPIX // L):
            best_v = jnp.full((L,), -1.0, jnp.float32)
            best_c = jnp.zeros((L,), jnp.int32)
            for cls in range(NUM_CLASSES):
                v = acc_v[pl.ds(cls * NPIX + c * L, L)]
                upd = v > best_v
                best_v = jnp.where(upd, v, best_v)
                best_c = jnp.where(upd, cls, best_c)
            pred_v[buf, pl.ds(c * L, L)] = best_c

        pltpu.sync_copy(pred_v.at[buf], out_hbm.at[base + i])

    def two_pairs(j, carry):
        process(2 * j, 0)
        process(jnp.minimum(2 * j + 1, PAIRS_PER_W - 1), 1)
        return carry

    lax.fori_loop(0, (PAIRS_PER_W + 1) // 2, two_pairs, 0)
    drain_in(0)


@functools.partial(
    pl.kernel,
    out_type=jax.ShapeDtypeStruct((NPAIR, NPIX), jnp.int32),
    mesh=plsc.VectorSubcoreMesh(core_axis_name="c", subcore_axis_name="s",
                                num_cores=2),
    compiler_params=pltpu.CompilerParams(needs_layout_passes=False),
    scratch_types=[
        pltpu.VMEM((2, K, NPIX), jnp.int32),
        pltpu.VMEM((2, K * L), jnp.float32),
        pltpu.VMEM((NUM_CLASSES * NPIX,), jnp.float32),
        pltpu.VMEM((2, NPIX), jnp.int32),
        pltpu.SemaphoreType.DMA,
        pltpu.SemaphoreType.DMA,
    ],
)
def _sc_vote_kernel(labc_hbm, wflat_hbm, out_hbm, lab_v, w_v, acc_v, pred_v,
                    in_sem, out_sem):
    _sc_vote(labc_hbm, wflat_hbm, out_hbm, lab_v, w_v, acc_v, pred_v,
             in_sem, out_sem)


def kernel(test_feature, train_features, train_labels):
    tf_t = jnp.transpose(test_feature, (1, 0, 2))
    labc, wexp = pl.pallas_call(
        _tc_body,
        grid=(P // G,),
        in_specs=[
            pl.BlockSpec((G, BS, D), lambda p: (p, 0, 0)),
            pl.BlockSpec((G, D, T), lambda p: (p, 0, 0)),
            pl.BlockSpec((G, NPIX, T), lambda p: (p, 0, 0)),
        ],
        out_specs=[
            pl.BlockSpec((G, BS * K, NPIX), lambda p: (p, 0, 0)),
            pl.BlockSpec((G, BS, K), lambda p: (p, 0, 0)),
        ],
        out_shape=[
            jax.ShapeDtypeStruct((P, BS * K, NPIX), jnp.int32),
            jax.ShapeDtypeStruct((P, BS, K), jnp.float32),
        ],
    )(tf_t, train_features, train_labels)

    wexp = jnp.broadcast_to(
        wexp.reshape(NPAIR, K, 1), (NPAIR, K, L)).astype(jnp.float32)
    pred = _sc_vote_kernel(
        labc.reshape(NPAIR, K, NPIX), wexp.reshape(NPAIR, K * L))

    img = jnp.transpose(pred.reshape(P, BS, NPIX), (1, 0, 2))
    img = img.reshape(BS, NROWS, NROWS, PS, PS)
    img = jnp.transpose(img, (0, 1, 3, 2, 4)).reshape(BS, NROWS * PS, NROWS * PS)
    return img


# speedup vs baseline: 1.2698x; 1.1276x over previous
import functools

import jax
import jax.numpy as jnp
from jax import lax
from jax.experimental import pallas as pl
from jax.experimental.pallas import tpu as pltpu
from jax.experimental.pallas import tpu_sc as plsc

BS = 8
P = 196
D = 384
T = 512
K = 20
NUM_CLASSES = 21
PS = 16
NPIX = PS * PS
NROWS = 14
G = 14

NPAIR = P * BS
NW = 32
PAIRS_PER_W = NPAIR // NW
L = 16


def _tc_body(tf_ref, trf_ref, lab_ref, labc_ref, wexp_ref):
    tf = tf_ref[...]
    trf = trf_ref[...]
    sim = lax.dot_general(
        tf, trf, (((2,), (1,)), ((0,), (0,))),
        preferred_element_type=jnp.float32,
    ).reshape(G * BS, T)

    cur = sim
    iota = lax.broadcasted_iota(jnp.int32, (G * BS, T), 1)
    masks = []
    ms = []
    for _ in range(K):
        m = jnp.max(cur, axis=1, keepdims=True)
        e = cur == m
        first = jnp.min(jnp.where(e, iota, T), axis=1, keepdims=True)
        e = iota == first
        masks.append(e.astype(jnp.float32))
        ms.append(m)
        cur = jnp.where(e, -jnp.inf, cur)
    S = jnp.stack(masks, axis=1)
    mk = jnp.concatenate(ms, axis=1)
    w = jax.nn.softmax(mk, axis=1)

    labf = lab_ref[...].astype(jnp.float32)
    labc = lax.dot_general(
        S.reshape(G, BS * K, T), labf, (((2,), (2,)), ((0,), (0,))),
        preferred_element_type=jnp.float32,
    )
    labc_ref[...] = labc.astype(jnp.int32)
    wexp_ref[...] = w.reshape(G, BS, K)


def _sc_vote(labc_hbm, wflat_hbm, out_hbm, lab_v, w_v, acc_v, pred_v,
             in_sem, out_sem):
    wid = lax.axis_index("s") * 2 + lax.axis_index("c")
    lane = lax.broadcasted_iota(jnp.int32, (L,), 0)
    base = wid * PAIRS_PER_W

    def start_in(i, buf):
        pltpu.async_copy(labc_hbm.at[base + i], lab_v.at[buf], in_sem)
        pltpu.async_copy(wflat_hbm.at[base + i], w_v.at[buf], in_sem)

    def drain_in(buf):
        pltpu.make_async_copy(labc_hbm.at[base], lab_v.at[buf], in_sem).wait()
        pltpu.make_async_copy(wflat_hbm.at[base], w_v.at[buf], in_sem).wait()

    start_in(0, 0)

    def process(i, buf):
        drain_in(buf)
        start_in(jnp.minimum(i + 1, PAIRS_PER_W - 1), 1 - buf)

        zero = jnp.zeros((L,), jnp.float32)
        for j in range(NUM_CLASSES * NPIX // L):
            acc_v[pl.ds(j * L, L)] = zero
        for k in range(K):
            wk = w_v[buf, pl.ds(k * L, L)]
            for c in range(NPIX // L):
                labk = lab_v[buf, k, pl.ds(c * L, L)]
                idx = labk * NPIX + (c * L + lane)
                plsc.addupdate_scatter(acc_v, [idx], wk)
        for c in range(NPIX // L):
            best_v = jnp.full((L,), -1.0, jnp.float32)
            best_c = jnp.zeros((L,), jnp.int32)
            for cls in range(NUM_CLASSES):
                v = acc_v[pl.ds(cls * NPIX + c * L, L)]
                upd = v > best_v
                best_v = jnp.where(upd, v, best_v)
                best_c = jnp.where(upd, cls, best_c)
            pred_v[buf, pl.ds(c * L, L)] = best_c

        pltpu.sync_copy(pred_v.at[buf], out_hbm.at[base + i])

    def two_pairs(j, carry):
        process(2 * j, 0)
        process(jnp.minimum(2 * j + 1, PAIRS_PER_W - 1), 1)
        return carry

    lax.fori_loop(0, (PAIRS_PER_W + 1) // 2, two_pairs, 0)
    drain_in(0)


@functools.partial(
    pl.kernel,
    out_type=jax.ShapeDtypeStruct((NPAIR, NPIX), jnp.int32),
    mesh=plsc.VectorSubcoreMesh(core_axis_name="c", subcore_axis_name="s",
                                num_cores=2),
    compiler_params=pltpu.CompilerParams(needs_layout_passes=False),
    scratch_types=[
        pltpu.VMEM((2, K, NPIX), jnp.int32),
        pltpu.VMEM((2, K * L), jnp.float32),
        pltpu.VMEM((NUM_CLASSES * NPIX,), jnp.float32),
        pltpu.VMEM((2, NPIX), jnp.int32),
        pltpu.SemaphoreType.DMA,
        pltpu.SemaphoreType.DMA,
    ],
)
def _sc_vote_kernel(labc_hbm, wflat_hbm, out_hbm, lab_v, w_v, acc_v, pred_v,
                    in_sem, out_sem):
    _sc_vote(labc_hbm, wflat_hbm, out_hbm, lab_v, w_v, acc_v, pred_v,
             in_sem, out_sem)


def kernel(test_feature, train_features, train_labels):
    tf_t = jnp.transpose(test_feature, (1, 0, 2))
    labc, wexp = pl.pallas_call(
        _tc_body,
        grid=(P // G,),
        in_specs=[
            pl.BlockSpec((G, BS, D), lambda p: (p, 0, 0)),
            pl.BlockSpec((G, D, T), lambda p: (p, 0, 0)),
            pl.BlockSpec((G, NPIX, T), lambda p: (p, 0, 0)),
        ],
        out_specs=[
            pl.BlockSpec((G, BS * K, NPIX), lambda p: (p, 0, 0)),
            pl.BlockSpec((G, BS, K), lambda p: (p, 0, 0)),
        ],
        out_shape=[
            jax.ShapeDtypeStruct((P, BS * K, NPIX), jnp.int32),
            jax.ShapeDtypeStruct((P, BS, K), jnp.float32),
        ],
    )(tf_t, train_features, train_labels)

    wexp = jnp.broadcast_to(
        wexp.reshape(NPAIR, K, 1), (NPAIR, K, L)).astype(jnp.float32)
    pred = _sc_vote_kernel(
        labc.reshape(NPAIR, K, NPIX), wexp.reshape(NPAIR, K * L))

    img = jnp.transpose(pred.reshape(P, BS, NPIX), (1, 0, 2))
    img = img.reshape(BS, NROWS, NROWS, PS, PS)
    img = jnp.transpose(img, (0, 1, 3, 2, 4)).reshape(BS, NROWS * PS, NROWS * PS)
    return img
